# baseline (speedup 1.0000x reference)
import functools

import jax
import jax.numpy as jnp
from jax import lax
from jax.experimental import pallas as pl
from jax.experimental.pallas import tpu as pltpu
from jax.experimental.pallas import tpu_sc as plsc

_NC, _NS = 2, 16
_CB = 8
_L = 16


def _sc_lookup_normalize(wv, xi, b, h, d):
    nw = _NC * _NS
    n4, dw = wv.shape
    rows_b = b // nw
    ck = _CB * h
    mesh = plsc.VectorSubcoreMesh(core_axis_name="c", subcore_axis_name="s")

    @functools.partial(
        pl.kernel,
        mesh=mesh,
        out_type=jax.ShapeDtypeStruct((b, h, d), jnp.float32),
        scratch_types=[
            pltpu.VMEM((ck,), jnp.int32),
            pltpu.VMEM((ck,), jnp.int32),
            pltpu.VMEM((ck, dw), jnp.float32),
            pltpu.VMEM((ck, d), jnp.float32),
            pltpu.SemaphoreType.DMA,
        ],
        compiler_params=pltpu.CompilerParams(needs_layout_passes=False),
    )
    def body(w_hbm, i_hbm, o_hbm, idx_v, idx4_v, rows_v, sel_v, sem):
        wid = lax.axis_index("s") * _NC + lax.axis_index("c")
        bstart = wid * rows_b

        @pl.loop(0, rows_b, step=_CB)
        def _(cb):
            batch0 = bstart + cb
            sync = pltpu.sync_copy
            sync(i_hbm.at[pl.ds(batch0 * h, ck)], idx_v)

            @pl.loop(0, ck, step=_L)
            def _(i):
                idx4_v[pl.ds(i, _L)] = idx_v[pl.ds(i, _L)] % n4

            pltpu.async_copy(w_hbm.at[idx4_v], rows_v, sem).wait()

            @pl.loop(0, ck, step=_L)
            def _(r0):
                iv = idx_v[pl.ds(r0, _L)]
                seg32 = (iv // n4) * d
                rowi = lax.iota(jnp.int32, _L) + r0
                acc = jnp.zeros((_L,), jnp.float32)
                for j in range(d):
                    vj = plsc.load_gather(rows_v, [rowi, seg32 + j])
                    acc = acc + vj * vj
                y = plsc.bitcast(0x5F3759DF - (plsc.bitcast(acc, jnp.int32) // 2),
                                 jnp.float32)
                hf = acc * jnp.float32(0.5)
                for _ in range(3):
                    y = y * (jnp.float32(1.5) - hf * y * y)
                rinv = jnp.minimum(y, 1.0e12)
                for j in range(d):
                    vj = plsc.load_gather(rows_v, [rowi, seg32 + j])
                    plsc.store_scatter(
                        sel_v,
                        [rowi, jnp.full((_L,), j, jnp.int32)],
                        vj * rinv)

            waits = [
                pltpu.async_copy(sel_v.at[pl.ds(t * h, h)],
                                 o_hbm.at[batch0 + t], sem)
                for t in range(_CB)
            ]
            for w in waits:
                w.wait()

    return body(wv, xi)


def kernel(x, weight):
    b, h = x.shape
    n, d = weight.shape
    xi = x.astype(jnp.int32).reshape(b * h)
    n4 = n // 4
    wv = jnp.concatenate(
        [weight[k * n4:(k + 1) * n4] for k in range(4)], axis=1)
    return _sc_lookup_normalize(wv, xi, b, h, d)


# speedup vs baseline: 1.1030x; 1.1030x over previous
import functools

import jax
import jax.numpy as jnp
from jax import lax
from jax.experimental import pallas as pl
from jax.experimental.pallas import tpu as pltpu
from jax.experimental.pallas import tpu_sc as plsc

_NC, _NS = 2, 16
_CB = 8
_L = 16


def _sc_lookup_normalize(wv, xi, b, h, d):
    nw = _NC * _NS
    n4, dw = wv.shape
    rows_b = b // nw
    ck = _CB * h
    mesh = plsc.VectorSubcoreMesh(core_axis_name="c", subcore_axis_name="s")

    @functools.partial(
        pl.kernel,
        mesh=mesh,
        out_type=jax.ShapeDtypeStruct((b, h, d), jnp.float32),
        scratch_types=[
            pltpu.VMEM((ck,), jnp.int32),
            pltpu.VMEM((ck,), jnp.int32),
            pltpu.VMEM((ck, dw), jnp.float32),
            pltpu.VMEM((ck, d), jnp.float32),
            pltpu.SemaphoreType.DMA,
        ],
        compiler_params=pltpu.CompilerParams(needs_layout_passes=False),
    )
    def body(w_hbm, i_hbm, o_hbm, idx_v, idx4_v, rows_v, sel_v, sem):
        wid = lax.axis_index("s") * _NC + lax.axis_index("c")
        bstart = wid * rows_b

        @pl.loop(0, rows_b, step=_CB)
        def _(cb):
            batch0 = bstart + cb
            sync = pltpu.sync_copy
            sync(i_hbm.at[pl.ds(batch0 * h, ck)], idx_v)

            @pl.loop(0, ck, step=_L)
            def _(i):
                idx4_v[pl.ds(i, _L)] = idx_v[pl.ds(i, _L)] // 4

            pltpu.async_copy(w_hbm.at[idx4_v], rows_v, sem).wait()

            @pl.loop(0, ck, step=_L)
            def _(r0):
                iv = idx_v[pl.ds(r0, _L)]
                seg32 = (iv % 4) * d
                rowi = lax.iota(jnp.int32, _L) + r0
                acc = jnp.zeros((_L,), jnp.float32)
                for j in range(d):
                    vj = plsc.load_gather(rows_v, [rowi, seg32 + j])
                    acc = acc + vj * vj
                y = plsc.bitcast(0x5F3759DF - (plsc.bitcast(acc, jnp.int32) // 2),
                                 jnp.float32)
                hf = acc * jnp.float32(0.5)
                for _ in range(3):
                    y = y * (jnp.float32(1.5) - hf * y * y)
                rinv = jnp.minimum(y, 1.0e12)
                for j in range(d):
                    vj = plsc.load_gather(rows_v, [rowi, seg32 + j])
                    plsc.store_scatter(
                        sel_v,
                        [rowi, jnp.full((_L,), j, jnp.int32)],
                        vj * rinv)

            waits = [
                pltpu.async_copy(sel_v.at[pl.ds(t * h, h)],
                                 o_hbm.at[batch0 + t], sem)
                for t in range(_CB)
            ]
            for w in waits:
                w.wait()

    return body(wv, xi)


def kernel(x, weight):
    b, h = x.shape
    n, d = weight.shape
    xi = x.astype(jnp.int32).reshape(b * h)
    wv = weight.reshape(n // 4, 4 * d)
    return _sc_lookup_normalize(wv, xi, b, h, d)


# speedup vs baseline: 1.2518x; 1.1349x over previous
import functools

import jax
import jax.numpy as jnp
from jax import lax
from jax.experimental import pallas as pl
from jax.experimental.pallas import tpu as pltpu
from jax.experimental.pallas import tpu_sc as plsc

_NC, _NS = 2, 16
_CHUNK = 800
_ROWS = 3200


def _sc_gather(wv, idx4):
    num_idx = idx4.shape[0]
    dw = wv.shape[1]
    nw = _NC * _NS
    b_per_w = num_idx // nw
    mesh = plsc.VectorSubcoreMesh(core_axis_name="c", subcore_axis_name="s")

    @functools.partial(
        pl.kernel,
        mesh=mesh,
        out_type=jax.ShapeDtypeStruct((num_idx, dw), wv.dtype),
        scratch_types=[
            pltpu.VMEM((_CHUNK,), jnp.int32),
            pltpu.VMEM((_CHUNK, dw), jnp.float32),
            pltpu.SemaphoreType.DMA,
        ],
    )
    def gather_kernel(w_hbm, i_hbm, o_hbm, idx_v, rows_v, sem):
        wid = lax.axis_index("s") * _NC + lax.axis_index("c")
        base = wid * b_per_w

        @pl.loop(0, b_per_w, step=_CHUNK)
        def _(off):
            pltpu.sync_copy(i_hbm.at[pl.ds(base + off, _CHUNK)], idx_v)
            pltpu.async_copy(w_hbm.at[idx_v], rows_v, sem).wait()
            pltpu.sync_copy(rows_v, o_hbm.at[pl.ds(base + off, _CHUNK)])

    return gather_kernel(wv, idx4)


_BB = 256


def _select_normalize(g, qw, b, h, d):
    n, dw = g.shape
    nsub = dw // d
    rows = _BB * h

    def body(g_ref, q_ref, o_ref):
        gb = g_ref[...]
        qb = q_ref[...]
        acc = jnp.zeros((rows, d), jnp.float32)
        for k in range(nsub):
            acc = jnp.where(qb == float(k), gb[:, k * d:(k + 1) * d], acc)
        s = jax.lax.dot_general(
            acc * acc, jnp.ones((d, 1), jnp.float32),
            (((1,), (0,)), ((), ())), preferred_element_type=jnp.float32)
        acc = acc / jnp.maximum(jnp.sqrt(s), 1e-12)
        for p in range(_BB):
            o_ref[p, :, :] = acc[p * h:(p + 1) * h, :]

    return pl.pallas_call(
        body,
        grid=(b // _BB,),
        in_specs=[
            pl.BlockSpec((rows, dw), lambda i: (i, 0)),
            pl.BlockSpec((rows, d), lambda i: (i, 0)),
        ],
        out_specs=pl.BlockSpec((_BB, h, d), lambda i: (i, 0, 0)),
        out_shape=jax.ShapeDtypeStruct((b, h, d), jnp.float32),
    )(g, qw)


def kernel(x, weight):
    b, h = x.shape
    n, d = weight.shape
    num_idx = b * h
    xi = x.astype(jnp.int32).reshape(num_idx)
    idx4 = xi // 4
    qw = jnp.broadcast_to(
        (xi % 4).astype(jnp.float32).reshape(num_idx, 1), (num_idx, d))
    wv = weight.reshape(n // 4, 4 * d)
    g = _sc_gather(wv, idx4)
    return _select_normalize(g, qw, b, h, d)
